# baseline (speedup 1.0000x reference)
import functools
import math

import jax
import jax.numpy as jnp
from jax import lax
from jax.experimental import pallas as pl
from jax.experimental.pallas import tpu as pltpu
from jax.experimental.pallas import tpu_sc as plsc

_N_LEVELS = 16
_F = 2
_HASHMAP = 2 ** 19
_BASE = 16
_MAXR = 512
_PLS = math.exp(math.log(_MAXR / _BASE) / (_N_LEVELS - 1))
_RES = []
_OFF = [0]
_tot = 0
for _i in range(_N_LEVELS):
    _r = math.floor(_BASE * _PLS ** _i)
    _RES.append(_r)
    _tot += min(_HASHMAP, (_r + 1) ** 3)
    _OFF.append(_tot)
_N_TOTAL = _tot
_P1 = 2654435761
_P2 = 805459861

_B = 262144
_NW = 32
_PPW = _B // _NW
_CHUNK = 128
_NCHUNK = _PPW // _CHUNK
_NGRP = _CHUNK // 16
_OUTC = 3 + 2 * _N_LEVELS


def _body(xh, yh, zh, emb_hbm, out_hbm,
          x_v, y_v, z_v,
          idx0, idx1, idx2, idx3, idx4, idx5, idx6, idx7,
          r0, r1, r2, r3, r4, r5, r6, r7,
          fx_v, fy_v, fz_v, outb_v, sem):
    idx_refs = (idx0, idx1, idx2, idx3, idx4, idx5, idx6, idx7)
    row_refs = (r0, r1, r2, r3, r4, r5, r6, r7)
    coord_refs = (x_v, y_v, z_v)
    frac_refs = (fx_v, fy_v, fz_v)
    nc = 2
    wid = lax.axis_index("s") * nc + lax.axis_index("c")
    base = wid * _PPW
    pltpu.sync_copy(xh.at[pl.ds(base, _PPW)], x_v)
    pltpu.sync_copy(yh.at[pl.ds(base, _PPW)], y_v)
    pltpu.sync_copy(zh.at[pl.ds(base, _PPW)], z_v)
    iota = lax.iota(jnp.int32, 16)

    def chunk_body(ch, carry):
        cb = ch * _CHUNK

        def xyz_store(g, c2):
            rbase = (g * 16 + iota) * _OUTC
            for d in range(3):
                v = coord_refs[d][pl.ds(cb + g * 16, 16)]
                plsc.store_scatter(outb_v, [rbase + d], v)
            return c2

        lax.fori_loop(0, _NGRP, xyz_store, 0)

        for lvl in range(_N_LEVELS):
            res = _RES[lvl]
            size = _OFF[lvl + 1] - _OFF[lvl]
            off0 = _OFF[lvl]
            resf = jnp.float32(res)
            resm1 = jnp.int32(res - 1)

            def idx_body(g, c2, resf=resf, resm1=resm1,
                         size=size, off0=off0):
                s = cb + g * 16
                hparts = []
                for d, prime in ((0, 1), (1, _P1), (2, _P2)):
                    u = coord_refs[d][pl.ds(s, 16)]
                    p = u * resf
                    vi = jnp.minimum(p.astype(jnp.int32), resm1)
                    fr = p - vi.astype(jnp.float32)
                    frac_refs[d][pl.ds(g * 16, 16)] = fr
                    uv = vi.astype(jnp.uint32)
                    h0 = uv * jnp.uint32(prime)
                    h1 = h0 + jnp.uint32(prime)
                    hparts.append((h0, h1))
                for c in range(8):
                    hx = hparts[0][(c >> 2) & 1]
                    hy = hparts[1][(c >> 1) & 1]
                    hz = hparts[2][c & 1]
                    h = hx ^ hy ^ hz
                    if size == _HASHMAP:
                        hm = h & jnp.uint32(size - 1)
                    else:
                        hm = h % jnp.uint32(size)
                    idx_refs[c][pl.ds(g * 16, 16)] = (
                        hm.astype(jnp.int32) + jnp.int32(off0))
                return c2

            lax.fori_loop(0, _NGRP, idx_body, 0)

            for c in range(8):
                pltpu.async_copy(
                    emb_hbm.at[idx_refs[c]], row_refs[c], sem).wait()

            def mac_body(g, c2, lvl=lvl):
                ridx = g * 16 + iota
                fx = fx_v[pl.ds(g * 16, 16)]
                fy = fy_v[pl.ds(g * 16, 16)]
                fz = fz_v[pl.ds(g * 16, 16)]
                gx = 1.0 - fx
                gy = 1.0 - fy
                gz = 1.0 - fz
                wxy = (gx * gy, gx * fy, fx * gy, fx * fy)
                acc0 = jnp.zeros((16,), jnp.float32)
                acc1 = jnp.zeros((16,), jnp.float32)
                zero16 = jnp.zeros((16,), jnp.int32)
                one16 = jnp.ones((16,), jnp.int32)
                for c in range(8):
                    wc = wxy[c >> 1] * (fz if (c & 1) else gz)
                    e0 = plsc.load_gather(row_refs[c], [ridx, zero16])
                    e1 = plsc.load_gather(row_refs[c], [ridx, one16])
                    acc0 = acc0 + wc * e0
                    acc1 = acc1 + wc * e1
                rbase = ridx * _OUTC
                plsc.store_scatter(outb_v, [rbase + (3 + 2 * lvl)], acc0)
                plsc.store_scatter(outb_v, [rbase + (4 + 2 * lvl)], acc1)
                return c2

            lax.fori_loop(0, _NGRP, mac_body, 0)

        pltpu.sync_copy(
            outb_v,
            out_hbm.at[pl.ds((base + cb) * _OUTC, _CHUNK * _OUTC)])
        return carry

    lax.fori_loop(0, _NCHUNK, chunk_body, 0)


_mesh = plsc.VectorSubcoreMesh(core_axis_name="c", subcore_axis_name="s")

_grid_kernel = functools.partial(
    pl.kernel,
    mesh=_mesh,
    compiler_params=pltpu.CompilerParams(
        needs_layout_passes=False, use_tc_tiling_on_sc=False),
    out_type=jax.ShapeDtypeStruct((_B * _OUTC,), jnp.float32),
    scratch_types=(
        [pltpu.VMEM((_PPW,), jnp.float32)] * 3
        + [pltpu.VMEM((_CHUNK,), jnp.int32)] * 8
        + [pltpu.VMEM((_CHUNK, 8), jnp.float32)] * 8
        + [pltpu.VMEM((_CHUNK,), jnp.float32)] * 3
        + [pltpu.VMEM((_CHUNK * _OUTC,), jnp.float32),
           pltpu.SemaphoreType.DMA]
    ),
)(_body)


def kernel(xyz, embeddings):
    x = xyz[:, 0]
    y = xyz[:, 1]
    z = xyz[:, 2]
    emb8 = jnp.pad(embeddings, ((0, 0), (0, 8 - _F)))
    flat = _grid_kernel(x, y, z, emb8)
    return flat.reshape(_B, _OUTC)


# speedup vs baseline: 1.5301x; 1.5301x over previous
import functools
import math

import jax
import jax.numpy as jnp
from jax import lax
from jax.experimental import pallas as pl
from jax.experimental.pallas import tpu as pltpu
from jax.experimental.pallas import tpu_sc as plsc

_N_LEVELS = 16
_F = 2
_HASHMAP = 2 ** 19
_BASE = 16
_MAXR = 512
_PLS = math.exp(math.log(_MAXR / _BASE) / (_N_LEVELS - 1))
_RES = []
_OFF = [0]
_tot = 0
for _i in range(_N_LEVELS):
    _r = math.floor(_BASE * _PLS ** _i)
    _RES.append(_r)
    _tot += min(_HASHMAP, (_r + 1) ** 3)
    _OFF.append(_tot)
_N_TOTAL = _tot
_P1 = 2654435761
_P2 = 805459861

_B = 262144
_NW = 32
_PPW = _B // _NW
_CHUNK = 128
_NCHUNK = _PPW // _CHUNK
_NGRP = _CHUNK // 16
_OUTC = 3 + 2 * _N_LEVELS


def _body(xh, yh, zh, emb_hbm, out_hbm,
          x_v, y_v, z_v,
          idxA, idxB, rowsA, rowsB,
          fxA, fyA, fzA, fxB, fyB, fzB,
          outb_v, semA, semB):
    idx_bufs = (idxA, idxB)
    row_bufs = (rowsA, rowsB)
    frac_bufs = ((fxA, fyA, fzA), (fxB, fyB, fzB))
    sems = (semA, semB)
    coord_refs = (x_v, y_v, z_v)
    nc = 2
    wid = lax.axis_index("s") * nc + lax.axis_index("c")
    base = wid * _PPW
    pltpu.sync_copy(xh.at[pl.ds(base, _PPW)], x_v)
    pltpu.sync_copy(yh.at[pl.ds(base, _PPW)], y_v)
    pltpu.sync_copy(zh.at[pl.ds(base, _PPW)], z_v)
    iota = lax.iota(jnp.int32, 16)

    def idx_pass(cb, lvl):
        bsel = lvl % 2
        idx_v = idx_bufs[bsel]
        frac_refs = frac_bufs[bsel]
        resf = jnp.float32(_RES[lvl])
        resm1 = jnp.int32(_RES[lvl] - 1)
        size = _OFF[lvl + 1] - _OFF[lvl]
        off0 = _OFF[lvl]

        def body(g, c2):
            s = cb + g * 16
            hparts = []
            for d, prime in ((0, 1), (1, _P1), (2, _P2)):
                u = coord_refs[d][pl.ds(s, 16)]
                p = u * resf
                vi = jnp.minimum(p.astype(jnp.int32), resm1)
                fr = p - vi.astype(jnp.float32)
                frac_refs[d][pl.ds(g * 16, 16)] = fr
                uv = vi.astype(jnp.uint32)
                h0 = uv * jnp.uint32(prime)
                h1 = h0 + jnp.uint32(prime)
                hparts.append((h0, h1))
            for c in range(8):
                hx = hparts[0][(c >> 2) & 1]
                hy = hparts[1][(c >> 1) & 1]
                hz = hparts[2][c & 1]
                h = hx ^ hy ^ hz
                if size == _HASHMAP:
                    hm = h & jnp.uint32(size - 1)
                else:
                    hm = h % jnp.uint32(size)
                idx_v[c, pl.ds(g * 16, 16)] = (
                    hm.astype(jnp.int32) + jnp.int32(off0))
            return c2

        lax.fori_loop(0, _NGRP, body, 0)

    def fire(lvl):
        bsel = lvl % 2
        return [
            pltpu.async_copy(
                emb_hbm.at[idx_bufs[bsel].at[c]],
                row_bufs[bsel].at[c], sems[bsel])
            for c in range(8)
        ]

    def mac_pass(lvl):
        bsel = lvl % 2
        rows_v = row_bufs[bsel]
        fx_v, fy_v, fz_v = frac_bufs[bsel]

        def body(g, c2):
            ridx = g * 16 + iota
            fx = fx_v[pl.ds(g * 16, 16)]
            fy = fy_v[pl.ds(g * 16, 16)]
            fz = fz_v[pl.ds(g * 16, 16)]
            gx = 1.0 - fx
            gy = 1.0 - fy
            gz = 1.0 - fz
            wxy = (gx * gy, gx * fy, fx * gy, fx * fy)
            acc0 = jnp.zeros((16,), jnp.float32)
            acc1 = jnp.zeros((16,), jnp.float32)
            zero16 = jnp.zeros((16,), jnp.int32)
            one16 = jnp.ones((16,), jnp.int32)
            for c in range(8):
                wc = wxy[c >> 1] * (fz if (c & 1) else gz)
                cfull = jnp.full((16,), c, jnp.int32)
                e0 = plsc.load_gather(rows_v, [cfull, ridx, zero16])
                e1 = plsc.load_gather(rows_v, [cfull, ridx, one16])
                acc0 = acc0 + wc * e0
                acc1 = acc1 + wc * e1
            rbase = ridx * _OUTC
            plsc.store_scatter(outb_v, [rbase + (3 + 2 * lvl)], acc0)
            plsc.store_scatter(outb_v, [rbase + (4 + 2 * lvl)], acc1)
            return c2

        lax.fori_loop(0, _NGRP, body, 0)

    def chunk_body(ch, carry):
        cb = ch * _CHUNK

        def xyz_store(g, c2):
            rbase = (g * 16 + iota) * _OUTC
            for d in range(3):
                v = coord_refs[d][pl.ds(cb + g * 16, 16)]
                plsc.store_scatter(outb_v, [rbase + d], v)
            return c2

        lax.fori_loop(0, _NGRP, xyz_store, 0)

        idx_pass(cb, 0)
        cps = fire(0)
        for lvl in range(1, _N_LEVELS):
            idx_pass(cb, lvl)
            cps_next = fire(lvl)
            for cp in cps:
                cp.wait()
            mac_pass(lvl - 1)
            cps = cps_next
        for cp in cps:
            cp.wait()
        mac_pass(_N_LEVELS - 1)

        pltpu.sync_copy(
            outb_v,
            out_hbm.at[pl.ds((base + cb) * _OUTC, _CHUNK * _OUTC)])
        return carry

    lax.fori_loop(0, _NCHUNK, chunk_body, 0)


_mesh = plsc.VectorSubcoreMesh(core_axis_name="c", subcore_axis_name="s")

_grid_kernel = functools.partial(
    pl.kernel,
    mesh=_mesh,
    compiler_params=pltpu.CompilerParams(
        needs_layout_passes=False, use_tc_tiling_on_sc=False),
    out_type=jax.ShapeDtypeStruct((_B * _OUTC,), jnp.float32),
    scratch_types=(
        [pltpu.VMEM((_PPW,), jnp.float32)] * 3
        + [pltpu.VMEM((8, _CHUNK), jnp.int32)] * 2
        + [pltpu.VMEM((8, _CHUNK, 8), jnp.float32)] * 2
        + [pltpu.VMEM((_CHUNK,), jnp.float32)] * 6
        + [pltpu.VMEM((_CHUNK * _OUTC,), jnp.float32),
           pltpu.SemaphoreType.DMA, pltpu.SemaphoreType.DMA]
    ),
)(_body)


def kernel(xyz, embeddings):
    x = xyz[:, 0]
    y = xyz[:, 1]
    z = xyz[:, 2]
    emb8 = jnp.pad(embeddings, ((0, 0), (0, 8 - _F)))
    flat = _grid_kernel(x, y, z, emb8)
    return flat.reshape(_B, _OUTC)


# speedup vs baseline: 1.7997x; 1.1762x over previous
import functools
import math

import jax
import jax.numpy as jnp
from jax import lax
from jax.experimental import pallas as pl
from jax.experimental.pallas import tpu as pltpu
from jax.experimental.pallas import tpu_sc as plsc

_N_LEVELS = 16
_F = 2
_HASHMAP = 2 ** 19
_BASE = 16
_MAXR = 512
_PLS = math.exp(math.log(_MAXR / _BASE) / (_N_LEVELS - 1))
_RES = []
_OFF = [0]
_tot = 0
for _i in range(_N_LEVELS):
    _r = math.floor(_BASE * _PLS ** _i)
    _RES.append(_r)
    _tot += min(_HASHMAP, (_r + 1) ** 3)
    _OFF.append(_tot)
_N_TOTAL = _tot
_P1 = 2654435761
_P2 = 805459861

_B = 262144
_NW = 32
_PPW = _B // _NW
_CHUNK = 128
_NCHUNK = _PPW // _CHUNK
_NGRP = _CHUNK // 16
_OUTC = 3 + 2 * _N_LEVELS


def _body(xh, yh, zh, emb_hbm, out_hbm,
          x_v, y_v, z_v,
          idxA, idxB, subA, subB, rowsA, rowsB,
          fxA, fyA, fzA, fxB, fyB, fzB,
          outb_v, semA, semB):
    idx_bufs = (idxA, idxB)
    sub_bufs = (subA, subB)
    row_bufs = (rowsA, rowsB)
    frac_bufs = ((fxA, fyA, fzA), (fxB, fyB, fzB))
    sems = (semA, semB)
    coord_refs = (x_v, y_v, z_v)
    nc = 2
    wid = lax.axis_index("s") * nc + lax.axis_index("c")
    base = wid * _PPW
    pltpu.sync_copy(xh.at[pl.ds(base, _PPW)], x_v)
    pltpu.sync_copy(yh.at[pl.ds(base, _PPW)], y_v)
    pltpu.sync_copy(zh.at[pl.ds(base, _PPW)], z_v)
    iota = lax.iota(jnp.int32, 16)

    def idx_pass(cb, lvl):
        bsel = lvl % 2
        idx_v = idx_bufs[bsel]
        sub_v = sub_bufs[bsel]
        frac_refs = frac_bufs[bsel]
        resf = jnp.float32(_RES[lvl])
        resm1 = jnp.int32(_RES[lvl] - 1)
        size = _OFF[lvl + 1] - _OFF[lvl]
        off0 = _OFF[lvl]

        def body(g, c2):
            s = cb + g * 16
            hparts = []
            for d, prime in ((0, 1), (1, _P1), (2, _P2)):
                u = coord_refs[d][pl.ds(s, 16)]
                p = u * resf
                vi = jnp.minimum(p.astype(jnp.int32), resm1)
                fr = p - vi.astype(jnp.float32)
                frac_refs[d][pl.ds(g * 16, 16)] = fr
                uv = vi.astype(jnp.uint32)
                h0 = uv * jnp.uint32(prime)
                h1 = h0 + jnp.uint32(prime)
                hparts.append((h0, h1))
            for c in range(8):
                hx = hparts[0][(c >> 2) & 1]
                hy = hparts[1][(c >> 1) & 1]
                hz = hparts[2][c & 1]
                h = hx ^ hy ^ hz
                if size == _HASHMAP:
                    hm = h & jnp.uint32(size - 1)
                else:
                    hm = h % jnp.uint32(size)
                grow = hm.astype(jnp.int32) + jnp.int32(off0)
                idx_v[c, pl.ds(g * 16, 16)] = lax.shift_right_logical(
                    grow, 2)
                sub_v[c, pl.ds(g * 16, 16)] = (grow + grow) & jnp.int32(6)
            return c2

        lax.fori_loop(0, _NGRP, body, 0)

    def fire(lvl):
        bsel = lvl % 2
        return [
            pltpu.async_copy(
                emb_hbm.at[idx_bufs[bsel].at[c]],
                row_bufs[bsel].at[c], sems[bsel])
            for c in range(8)
        ]

    def mac_pass(lvl):
        bsel = lvl % 2
        rows_v = row_bufs[bsel]
        sub_v = sub_bufs[bsel]
        fx_v, fy_v, fz_v = frac_bufs[bsel]

        def body(g, c2):
            ridx = g * 16 + iota
            fx = fx_v[pl.ds(g * 16, 16)]
            fy = fy_v[pl.ds(g * 16, 16)]
            fz = fz_v[pl.ds(g * 16, 16)]
            gx = 1.0 - fx
            gy = 1.0 - fy
            gz = 1.0 - fz
            wxy = (gx * gy, gx * fy, fx * gy, fx * fy)
            acc0 = jnp.zeros((16,), jnp.float32)
            acc1 = jnp.zeros((16,), jnp.float32)
            for c in range(8):
                wc = wxy[c >> 1] * (fz if (c & 1) else gz)
                cfull = jnp.full((16,), c, jnp.int32)
                col = sub_v[c, pl.ds(g * 16, 16)]
                e0 = plsc.load_gather(rows_v, [cfull, ridx, col])
                e1 = plsc.load_gather(rows_v, [cfull, ridx, col + 1])
                acc0 = acc0 + wc * e0
                acc1 = acc1 + wc * e1
            rbase = ridx * _OUTC
            plsc.store_scatter(outb_v, [rbase + (3 + 2 * lvl)], acc0)
            plsc.store_scatter(outb_v, [rbase + (4 + 2 * lvl)], acc1)
            return c2

        lax.fori_loop(0, _NGRP, body, 0)

    def chunk_body(ch, carry):
        cb = ch * _CHUNK

        def xyz_store(g, c2):
            rbase = (g * 16 + iota) * _OUTC
            for d in range(3):
                v = coord_refs[d][pl.ds(cb + g * 16, 16)]
                plsc.store_scatter(outb_v, [rbase + d], v)
            return c2

        lax.fori_loop(0, _NGRP, xyz_store, 0)

        idx_pass(cb, 0)
        cps = fire(0)
        for lvl in range(1, _N_LEVELS):
            idx_pass(cb, lvl)
            cps_next = fire(lvl)
            for cp in cps:
                cp.wait()
            mac_pass(lvl - 1)
            cps = cps_next
        for cp in cps:
            cp.wait()
        mac_pass(_N_LEVELS - 1)

        pltpu.sync_copy(
            outb_v,
            out_hbm.at[pl.ds((base + cb) * _OUTC, _CHUNK * _OUTC)])
        return carry

    lax.fori_loop(0, _NCHUNK, chunk_body, 0)


_mesh = plsc.VectorSubcoreMesh(core_axis_name="c", subcore_axis_name="s")

_grid_kernel = functools.partial(
    pl.kernel,
    mesh=_mesh,
    compiler_params=pltpu.CompilerParams(
        needs_layout_passes=False, use_tc_tiling_on_sc=False),
    out_type=jax.ShapeDtypeStruct((_B * _OUTC,), jnp.float32),
    scratch_types=(
        [pltpu.VMEM((_PPW,), jnp.float32)] * 3
        + [pltpu.VMEM((8, _CHUNK), jnp.int32)] * 4
        + [pltpu.VMEM((8, _CHUNK, 8), jnp.float32)] * 2
        + [pltpu.VMEM((_CHUNK,), jnp.float32)] * 6
        + [pltpu.VMEM((_CHUNK * _OUTC,), jnp.float32),
           pltpu.SemaphoreType.DMA, pltpu.SemaphoreType.DMA]
    ),
)(_body)


def kernel(xyz, embeddings):
    x = xyz[:, 0]
    y = xyz[:, 1]
    z = xyz[:, 2]
    emb_blk = embeddings.reshape(_N_TOTAL // 4, 8)
    flat = _grid_kernel(x, y, z, emb_blk)
    return flat.reshape(_B, _OUTC)


# speedup vs baseline: 4.7648x; 2.6476x over previous
import functools
import math

import jax
import jax.numpy as jnp
from jax import lax
from jax.experimental import pallas as pl
from jax.experimental.pallas import tpu as pltpu
from jax.experimental.pallas import tpu_sc as plsc

_N_LEVELS = 16
_F = 2
_HASHMAP = 2 ** 19
_BASE = 16
_MAXR = 512
_PLS = math.exp(math.log(_MAXR / _BASE) / (_N_LEVELS - 1))
_RES = []
_OFF = [0]
_tot = 0
for _i in range(_N_LEVELS):
    _r = math.floor(_BASE * _PLS ** _i)
    _RES.append(_r)
    _tot += min(_HASHMAP, (_r + 1) ** 3)
    _OFF.append(_tot)
_N_TOTAL = _tot
_P1 = 2654435761
_P2 = 805459861

_B = 262144
_NW = 32
_PPW = _B // _NW
_CHUNK = 128
_NCHUNK = _PPW // _CHUNK
_NGRP = _CHUNK // 16
_OUTC = 3 + 2 * _N_LEVELS
_NPAD = -(-_N_TOTAL // 8) * 8
_MBLK = _NPAD // 8


def _body(xh, yh, zh, e0_hbm, e1_hbm, out_hbm,
          x_v, y_v, z_v,
          idxA, idxB, subA, subB,
          rows0A, rows0B, rows1A, rows1B,
          fxA, fyA, fzA, fxB, fyB, fzB,
          outb_v, semA, semB):
    idx_bufs = (idxA, idxB)
    sub_bufs = (subA, subB)
    row0_bufs = (rows0A, rows0B)
    row1_bufs = (rows1A, rows1B)
    frac_bufs = ((fxA, fyA, fzA), (fxB, fyB, fzB))
    sems = (semA, semB)
    coord_refs = (x_v, y_v, z_v)
    nc = 2
    wid = lax.axis_index("s") * nc + lax.axis_index("c")
    base = wid * _PPW
    pltpu.sync_copy(xh.at[pl.ds(base, _PPW)], x_v)
    pltpu.sync_copy(yh.at[pl.ds(base, _PPW)], y_v)
    pltpu.sync_copy(zh.at[pl.ds(base, _PPW)], z_v)
    iota = lax.iota(jnp.int32, 16)

    def idx_pass(cb, lvl):
        bsel = lvl % 2
        idx_v = idx_bufs[bsel]
        sub_v = sub_bufs[bsel]
        frac_refs = frac_bufs[bsel]
        resf = jnp.float32(_RES[lvl])
        resm1 = jnp.int32(_RES[lvl] - 1)
        size = _OFF[lvl + 1] - _OFF[lvl]
        off0 = _OFF[lvl]

        def body(g, c2):
            s = cb + g * 16
            hparts = []
            for d, prime in ((0, 1), (1, _P1), (2, _P2)):
                u = coord_refs[d][pl.ds(s, 16)]
                p = u * resf
                vi = jnp.minimum(p.astype(jnp.int32), resm1)
                fr = p - vi.astype(jnp.float32)
                frac_refs[d][pl.ds(g * 16, 16)] = fr
                uv = vi.astype(jnp.uint32)
                h0 = uv * jnp.uint32(prime)
                h1 = h0 + jnp.uint32(prime)
                hparts.append((h0, h1))
            for c in range(8):
                hx = hparts[0][(c >> 2) & 1]
                hy = hparts[1][(c >> 1) & 1]
                hz = hparts[2][c & 1]
                h = hx ^ hy ^ hz
                if size == _HASHMAP:
                    hm = h & jnp.uint32(size - 1)
                else:
                    hm = h % jnp.uint32(size)
                grow = hm.astype(jnp.int32) + jnp.int32(off0)
                idx_v[c, pl.ds(g * 16, 16)] = lax.shift_right_logical(
                    grow, 3)
                sub_v[c, pl.ds(g * 16, 16)] = grow & jnp.int32(7)
            return c2

        lax.fori_loop(0, _NGRP, body, 0)

    def fire(lvl):
        bsel = lvl % 2
        cps = []
        for c in range(8):
            cps.append(pltpu.async_copy(
                e0_hbm.at[idx_bufs[bsel].at[c]],
                row0_bufs[bsel].at[c], sems[bsel]))
            cps.append(pltpu.async_copy(
                e1_hbm.at[idx_bufs[bsel].at[c]],
                row1_bufs[bsel].at[c], sems[bsel]))
        return cps

    def mac_pass(lvl):
        bsel = lvl % 2
        rows0_v = row0_bufs[bsel]
        rows1_v = row1_bufs[bsel]
        sub_v = sub_bufs[bsel]
        fx_v, fy_v, fz_v = frac_bufs[bsel]

        def body(g, c2):
            ridx = g * 16 + iota
            fx = fx_v[pl.ds(g * 16, 16)]
            fy = fy_v[pl.ds(g * 16, 16)]
            fz = fz_v[pl.ds(g * 16, 16)]
            gx = 1.0 - fx
            gy = 1.0 - fy
            gz = 1.0 - fz
            wxy = (gx * gy, gx * fy, fx * gy, fx * fy)
            acc0 = jnp.zeros((16,), jnp.float32)
            acc1 = jnp.zeros((16,), jnp.float32)
            for c in range(8):
                wc = wxy[c >> 1] * (fz if (c & 1) else gz)
                cfull = jnp.full((16,), c, jnp.int32)
                col = sub_v[c, pl.ds(g * 16, 16)]
                e0 = plsc.load_gather(rows0_v, [cfull, ridx, col])
                e1 = plsc.load_gather(rows1_v, [cfull, ridx, col])
                acc0 = acc0 + wc * e0
                acc1 = acc1 + wc * e1
            rbase = ridx * _OUTC
            plsc.store_scatter(outb_v, [rbase + (3 + 2 * lvl)], acc0)
            plsc.store_scatter(outb_v, [rbase + (4 + 2 * lvl)], acc1)
            return c2

        lax.fori_loop(0, _NGRP, body, 0)

    def chunk_body(ch, carry):
        cb = ch * _CHUNK

        def xyz_store(g, c2):
            rbase = (g * 16 + iota) * _OUTC
            for d in range(3):
                v = coord_refs[d][pl.ds(cb + g * 16, 16)]
                plsc.store_scatter(outb_v, [rbase + d], v)
            return c2

        lax.fori_loop(0, _NGRP, xyz_store, 0)

        idx_pass(cb, 0)
        cps = fire(0)
        for lvl in range(1, _N_LEVELS):
            idx_pass(cb, lvl)
            cps_next = fire(lvl)
            for cp in cps:
                cp.wait()
            mac_pass(lvl - 1)
            cps = cps_next
        for cp in cps:
            cp.wait()
        mac_pass(_N_LEVELS - 1)

        pltpu.sync_copy(
            outb_v,
            out_hbm.at[pl.ds((base + cb) * _OUTC, _CHUNK * _OUTC)])
        return carry

    lax.fori_loop(0, _NCHUNK, chunk_body, 0)


_mesh = plsc.VectorSubcoreMesh(core_axis_name="c", subcore_axis_name="s")

_grid_kernel = functools.partial(
    pl.kernel,
    mesh=_mesh,
    compiler_params=pltpu.CompilerParams(
        needs_layout_passes=False, use_tc_tiling_on_sc=False),
    out_type=jax.ShapeDtypeStruct((_B * _OUTC,), jnp.float32),
    scratch_types=(
        [pltpu.VMEM((_PPW,), jnp.float32)] * 3
        + [pltpu.VMEM((8, _CHUNK), jnp.int32)] * 4
        + [pltpu.VMEM((8, _CHUNK, 8), jnp.float32)] * 4
        + [pltpu.VMEM((_CHUNK,), jnp.float32)] * 6
        + [pltpu.VMEM((_CHUNK * _OUTC,), jnp.float32),
           pltpu.SemaphoreType.DMA, pltpu.SemaphoreType.DMA]
    ),
)(_body)


def kernel(xyz, embeddings):
    x = xyz[:, 0]
    y = xyz[:, 1]
    z = xyz[:, 2]
    pad = _NPAD - _N_TOTAL
    e0 = jnp.pad(embeddings[:, 0], (0, pad)).reshape(_MBLK, 8)
    e1 = jnp.pad(embeddings[:, 1], (0, pad)).reshape(_MBLK, 8)
    flat = _grid_kernel(x, y, z, e0, e1)
    return flat.reshape(_B, _OUTC)
